# baseline (speedup 1.0000x reference)
import functools

import jax
import jax.numpy as jnp
from jax import lax
from jax.experimental import pallas as pl
from jax.experimental.pallas import tpu as pltpu
from jax.experimental.pallas import tpu_sc as plsc

VOCAB = 1000
ROWS = 1024 * 50
NC, NS = 2, 16
NW = NC * NS
RPW = ROWS // NW
BLK = 32
NBLK = RPW // BLK


@functools.partial(
    pl.kernel,
    out_type=jax.ShapeDtypeStruct((ROWS * VOCAB,), jnp.float32),
    mesh=plsc.VectorSubcoreMesh(
        core_axis_name="c", subcore_axis_name="s", num_cores=NC, num_subcores=NS
    ),
    scratch_types=[
        pltpu.VMEM((RPW,), jnp.int32),
        pltpu.VMEM((BLK * VOCAB,), jnp.float32),
        pltpu.VMEM((BLK * VOCAB,), jnp.float32),
        pltpu.SemaphoreType.DMA,
        pltpu.SemaphoreType.DMA,
    ],
    compiler_params=pltpu.CompilerParams(needs_layout_passes=False),
)
def _one_hot_sc(tok_hbm, out_hbm, tok_v, buf0, buf1, sem0, sem1):
    wid = lax.axis_index("s") * NC + lax.axis_index("c")
    base = wid * RPW
    bufs = (buf0, buf1)
    sems = (sem0, sem1)

    pltpu.sync_copy(tok_hbm.at[pl.ds(base, RPW)], tok_v)

    zeros = jnp.zeros((16,), jnp.float32)
    ones = jnp.ones((16,), jnp.float32)
    lane = lax.broadcasted_iota(jnp.int32, (16,), 0)

    def _memset(i, carry):
        for b in range(2):
            for k in range(8):
                bufs[b][pl.ds(i * 128 + k * 16, 16)] = zeros
        return carry

    lax.fori_loop(0, BLK * VOCAB // 128, _memset, None)

    def _scatter(buf, blk, vals):
        for g in range(BLK // 16):
            tv = tok_v[pl.ds(blk * BLK + g * 16, 16)]
            idx = (lane + g * 16) * VOCAB + tv
            plsc.store_scatter(buf, [idx], vals)

    def _dma_out(b, blk):
        dst = out_hbm.at[pl.ds((base + blk * BLK) * VOCAB, BLK * VOCAB)]
        pltpu.async_copy(bufs[b], dst, sems[b])

    def _dma_wait(b, blk):
        dst = out_hbm.at[pl.ds((base + blk * BLK) * VOCAB, BLK * VOCAB)]
        pltpu.make_async_copy(bufs[b], dst, sems[b]).wait()

    for b in range(2):
        _scatter(bufs[b], jnp.int32(b), ones)
        _dma_out(b, jnp.int32(b))

    def _step(i, carry):
        for b in range(2):
            blk = i * 2 + b
            _dma_wait(b, blk - 2)
            _scatter(bufs[b], blk - 2, zeros)
            _scatter(bufs[b], blk, ones)
            _dma_out(b, blk)
        return carry

    lax.fori_loop(1, NBLK // 2, _step, None)

    for b in range(2):
        _dma_wait(b, jnp.int32(NBLK - 2 + b))


def kernel(tokens):
    flat = tokens.reshape(-1).astype(jnp.int32)
    out = _one_hot_sc(flat)
    return out.reshape(tokens.shape + (VOCAB,))


# speedup vs baseline: 6.5866x; 6.5866x over previous
import functools

import jax
import jax.numpy as jnp
from jax import lax
from jax.experimental import pallas as pl
from jax.experimental.pallas import tpu as pltpu
from jax.experimental.pallas import tpu_sc as plsc

VOCAB = 1000
NS_TOK = 50
BATCH = 1024
NC, NS = 2, 16
VN = 256
NITEM = NS_TOK


@functools.partial(
    pl.kernel,
    out_type=jax.ShapeDtypeStruct((NS_TOK, VOCAB, BATCH), jnp.float32),
    mesh=plsc.VectorSubcoreMesh(
        core_axis_name="c", subcore_axis_name="s", num_cores=NC, num_subcores=NS
    ),
    scratch_types=[
        pltpu.VMEM((NS_TOK, 128), jnp.int32),
        pltpu.VMEM((VN, 128), jnp.float32),
        pltpu.VMEM((VN, 128), jnp.float32),
        pltpu.SemaphoreType.DMA,
        pltpu.SemaphoreType.DMA,
    ],
    compiler_params=pltpu.CompilerParams(
        needs_layout_passes=False, use_tc_tiling_on_sc=True
    ),
)
def _one_hot_sc(tok_hbm, out_hbm, tok_v, buf0, buf1, sem0, sem1):
    wid = lax.axis_index("s") * NC + lax.axis_index("c")
    tb = wid >> 2
    c = wid & 3
    v0 = jnp.where(c == 3, VOCAB - VN, c * VN)
    bufs = (buf0, buf1)
    sems = (sem0, sem1)

    pltpu.sync_copy(tok_hbm.at[:, pl.ds(tb * 128, 128)], tok_v)

    zeros = jnp.zeros((16,), jnp.float32)
    ones = jnp.ones((16,), jnp.float32)
    lane = lax.broadcasted_iota(jnp.int32, (16,), 0)

    def _memset_row(i, carry):
        for b in range(2):
            for k in range(8):
                bufs[b][i, pl.ds(k * 16, 16)] = zeros
        return carry

    lax.fori_loop(0, VN, _memset_row, None)

    def _scatter(buf, s, vals):
        for g in range(8):
            tv = tok_v[s, pl.ds(g * 16, 16)]
            mask = (tv >= v0) & (tv < v0 + VN)
            plsc.store_scatter(buf, [tv - v0, lane + g * 16], vals, mask=mask)

    def _dma_out(b, s):
        dst = out_hbm.at[s, pl.ds(v0, VN), pl.ds(tb * 128, 128)]
        pltpu.async_copy(bufs[b], dst, sems[b])

    def _dma_wait(b, s):
        dst = out_hbm.at[s, pl.ds(v0, VN), pl.ds(tb * 128, 128)]
        pltpu.make_async_copy(bufs[b], dst, sems[b]).wait()

    for b in range(2):
        _scatter(bufs[b], jnp.int32(b), ones)
        _dma_out(b, jnp.int32(b))

    def _step(i, carry):
        for b in range(2):
            s = i * 2 + b
            _dma_wait(b, s - 2)
            _scatter(bufs[b], s - 2, zeros)
            _scatter(bufs[b], s, ones)
            _dma_out(b, s)
        return carry

    lax.fori_loop(1, NITEM // 2, _step, None)

    for b in range(2):
        _dma_wait(b, jnp.int32(NITEM - 2 + b))


def kernel(tokens):
    tok_t = jnp.swapaxes(tokens.astype(jnp.int32), 0, 1)
    out = _one_hot_sc(tok_t)
    return jnp.transpose(out, (2, 0, 1))


# speedup vs baseline: 6.5882x; 1.0002x over previous
import functools

import jax
import jax.numpy as jnp
from jax import lax
from jax.experimental import pallas as pl
from jax.experimental.pallas import tpu as pltpu
from jax.experimental.pallas import tpu_sc as plsc

VOCAB = 1000
NS_TOK = 50
BATCH = 1024
NC, NS = 2, 16
VN = 256
NITEM = NS_TOK


@functools.partial(
    pl.kernel,
    out_type=jax.ShapeDtypeStruct((NS_TOK, VOCAB, BATCH), jnp.float32),
    mesh=plsc.VectorSubcoreMesh(
        core_axis_name="c", subcore_axis_name="s", num_cores=NC, num_subcores=NS
    ),
    scratch_types=[
        pltpu.VMEM((NS_TOK, 128), jnp.int32),
        pltpu.VMEM((VN, 128), jnp.float32),
        pltpu.VMEM((VN, 128), jnp.float32),
        pltpu.SemaphoreType.DMA,
        pltpu.SemaphoreType.DMA,
    ],
    compiler_params=pltpu.CompilerParams(
        needs_layout_passes=False,
        use_tc_tiling_on_sc=True,
        disable_bounds_checks=True,
        disable_semaphore_checks=True,
        skip_device_barrier=True,
    ),
)
def _one_hot_sc(tok_hbm, out_hbm, tok_v, buf0, buf1, sem0, sem1):
    wid = lax.axis_index("s") * NC + lax.axis_index("c")
    tb = wid >> 2
    c = wid & 3
    v0 = jnp.where(c == 3, VOCAB - VN, c * VN)
    bufs = (buf0, buf1)
    sems = (sem0, sem1)

    pltpu.sync_copy(tok_hbm.at[:, pl.ds(tb * 128, 128)], tok_v)

    zeros = jnp.zeros((16,), jnp.float32)
    ones = jnp.ones((16,), jnp.float32)
    lane = lax.broadcasted_iota(jnp.int32, (16,), 0)

    def _memset_row(i, carry):
        for b in range(2):
            for k in range(8):
                bufs[b][i, pl.ds(k * 16, 16)] = zeros
        return carry

    lax.fori_loop(0, VN, _memset_row, None)

    def _scatter(buf, s, vals):
        for g in range(8):
            tv = tok_v[s, pl.ds(g * 16, 16)]
            mask = (tv >= v0) & (tv < v0 + VN)
            plsc.store_scatter(buf, [tv - v0, lane + g * 16], vals, mask=mask)

    def _dma_out(b, s):
        dst = out_hbm.at[s, pl.ds(v0, VN), pl.ds(tb * 128, 128)]
        pltpu.async_copy(bufs[b], dst, sems[b])

    def _dma_wait(b, s):
        dst = out_hbm.at[s, pl.ds(v0, VN), pl.ds(tb * 128, 128)]
        pltpu.make_async_copy(bufs[b], dst, sems[b]).wait()

    for b in range(2):
        _scatter(bufs[b], jnp.int32(b), ones)
        _dma_out(b, jnp.int32(b))

    def _step(i, carry):
        for b in range(2):
            s = i * 2 + b
            _dma_wait(b, s - 2)
            _scatter(bufs[b], s - 2, zeros)
            _scatter(bufs[b], s, ones)
            _dma_out(b, s)
        return carry

    lax.fori_loop(1, NITEM // 2, _step, None)

    for b in range(2):
        _dma_wait(b, jnp.int32(NITEM - 2 + b))


def kernel(tokens):
    tok_t = jnp.swapaxes(tokens.astype(jnp.int32), 0, 1)
    out = _one_hot_sc(tok_t)
    return jnp.transpose(out, (2, 0, 1))


# speedup vs baseline: 6.6452x; 1.0087x over previous
import functools

import jax
import jax.numpy as jnp
from jax import lax
from jax.experimental import pallas as pl
from jax.experimental.pallas import tpu as pltpu
from jax.experimental.pallas import tpu_sc as plsc

VOCAB = 1000
NS_TOK = 50
BATCH = 1024
NC, NS = 2, 16
VN = 256
NITEM = NS_TOK


@functools.partial(
    pl.kernel,
    out_type=jax.ShapeDtypeStruct((NS_TOK, VOCAB, BATCH), jnp.float32),
    mesh=plsc.VectorSubcoreMesh(
        core_axis_name="c", subcore_axis_name="s", num_cores=NC, num_subcores=NS
    ),
    scratch_types=[
        pltpu.VMEM((NS_TOK, 128), jnp.int32),
        pltpu.VMEM((VN, 128), jnp.float32),
        pltpu.VMEM((VN, 128), jnp.float32),
        pltpu.SemaphoreType.DMA,
        pltpu.SemaphoreType.DMA,
    ],
    compiler_params=pltpu.CompilerParams(
        needs_layout_passes=False,
        use_tc_tiling_on_sc=True,
        disable_bounds_checks=True,
        disable_semaphore_checks=True,
        skip_device_barrier=True,
    ),
)
def _one_hot_sc(tok_hbm, out_hbm, tok_v, buf0, buf1, sem0, sem1):
    wid = lax.axis_index("s") * NC + lax.axis_index("c")
    tb = wid >> 2
    c = wid & 3
    v0 = jnp.where(c == 3, VOCAB - VN, c * VN)
    bufs = (buf0, buf1)
    sems = (sem0, sem1)

    pltpu.sync_copy(tok_hbm.at[:, pl.ds(tb * 128, 128)], tok_v)

    zeros = jnp.zeros((16,), jnp.float32)
    ones = jnp.ones((16,), jnp.float32)
    lane = lax.broadcasted_iota(jnp.int32, (16,), 0)

    def _memset(buf):
        def _row(i, carry):
            for k in range(8):
                buf[i, pl.ds(k * 16, 16)] = zeros
            return carry

        lax.fori_loop(0, VN, _row, None)

    def _scatter(buf, s, vals):
        for g in range(8):
            tv = tok_v[s, pl.ds(g * 16, 16)]
            mask = (tv >= v0) & (tv < v0 + VN)
            plsc.store_scatter(buf, [tv - v0, lane + g * 16], vals, mask=mask)

    def _dma_out(b, s):
        dst = out_hbm.at[s, pl.ds(v0, VN), pl.ds(tb * 128, 128)]
        pltpu.async_copy(bufs[b], dst, sems[b])

    def _dma_wait(b, s):
        dst = out_hbm.at[s, pl.ds(v0, VN), pl.ds(tb * 128, 128)]
        pltpu.make_async_copy(bufs[b], dst, sems[b]).wait()

    for b in range(2):
        _memset(bufs[b])
        _scatter(bufs[b], jnp.int32(b), ones)
        _dma_out(b, jnp.int32(b))

    def _step(i, carry):
        for b in range(2):
            s = i * 2 + b
            _dma_wait(b, s - 2)
            _scatter(bufs[b], s - 2, zeros)
            _scatter(bufs[b], s, ones)
            _dma_out(b, s)
        return carry

    lax.fori_loop(1, NITEM // 2, _step, None)

    for b in range(2):
        _dma_wait(b, jnp.int32(NITEM - 2 + b))


def kernel(tokens):
    tok_t = jnp.swapaxes(tokens.astype(jnp.int32), 0, 1)
    out = _one_hot_sc(tok_t)
    return jnp.transpose(out, (2, 0, 1))
